# Initial kernel scaffold; baseline (speedup 1.0000x reference)
import jax
import jax.numpy as jnp
from jax.experimental import pallas as pl


def kernel(x, vn_embedding, batch, W_vn2node, b_vn2node, g_vn2node, be_vn2node, W_node2vn, b_node2vn, g_node2vn, be_node2vn):
    raise NotImplementedError("write your pallas kernel here")


# speedup vs baseline: 9.9513x; 9.9513x over previous
import functools

import jax
import jax.numpy as jnp
from jax import lax
from jax.experimental import pallas as pl

_EPS = 1e-5


def _seg_kernel(batch_ref, x_ref, sums_ref, counts_ref):
    i = pl.program_id(0)

    @pl.when(i == 0)
    def _init():
        sums_ref[...] = jnp.zeros_like(sums_ref)
        counts_ref[...] = jnp.zeros_like(counts_ref)

    b = batch_ref[0]
    B = sums_ref.shape[0]
    RB = b.shape[-1]
    oh = (lax.broadcasted_iota(jnp.int32, (B, RB), 0)
          == jnp.broadcast_to(b, (B, RB))).astype(jnp.float32)
    sums_ref[...] += lax.dot(oh, x_ref[...], preferred_element_type=jnp.float32)
    counts_ref[...] += jnp.sum(oh, axis=1)[None, :]


def _mlp_kernel(n_rows, sums_ref, counts_ref, vn_ref,
                Wn_ref, bn_ref, gn_ref, ben_ref,
                Wv_ref, bv_ref, gv_ref, bev_ref,
                z_ref, vn_out_ref):
    counts = counts_ref[0, :]
    cnt = counts[:, None]
    ntv = sums_ref[...] / jnp.where(cnt > 0, cnt, 1.0)
    h = lax.dot_general(ntv, Wn_ref[...], (((1,), (1,)), ((), ())),
                        preferred_element_type=jnp.float32) + bn_ref[...]
    mu = jnp.mean(h, axis=0)
    var = jnp.mean((h - mu[None, :]) ** 2, axis=0)
    h = gn_ref[...] * (h - mu[None, :]) * lax.rsqrt(var[None, :] + _EPS) + ben_ref[...]
    h = jnp.maximum(h, 0.0)
    vn_out = vn_ref[...] + h
    vn_out_ref[...] = vn_out
    y = lax.dot_general(vn_out, Wv_ref[...], (((1,), (1,)), ((), ())),
                        preferred_element_type=jnp.float32) + bv_ref[...]
    w = (counts / jnp.float32(n_rows))[:, None]
    mu2 = jnp.sum(w * y, axis=0)
    var2 = jnp.sum(w * (y - mu2[None, :]) ** 2, axis=0)
    z = gv_ref[...] * (y - mu2[None, :]) * lax.rsqrt(var2[None, :] + _EPS) + bev_ref[...]
    z_ref[...] = jnp.maximum(z, 0.0)


def _bcast_kernel(batch_ref, x_ref, z_ref, out_ref):
    b = batch_ref[0]
    B = z_ref.shape[0]
    RC = b.shape[-1]
    oh = (lax.broadcasted_iota(jnp.int32, (B, RC), 0)
          == jnp.broadcast_to(b, (B, RC))).astype(jnp.float32)
    gathered = lax.dot_general(oh, z_ref[...], (((0,), (0,)), ((), ())),
                               preferred_element_type=jnp.float32)
    out_ref[...] = x_ref[...] + gathered


def kernel(x, vn_embedding, batch, W_vn2node, b_vn2node, g_vn2node, be_vn2node,
           W_node2vn, b_node2vn, g_node2vn, be_node2vn):
    N, D = x.shape
    B = vn_embedding.shape[0]
    batch = batch.astype(jnp.int32)

    RB = 2000
    nblk = N // RB
    assert nblk * RB == N
    batch3 = batch.reshape(nblk, 1, RB)

    sums, counts = pl.pallas_call(
        _seg_kernel,
        grid=(nblk,),
        in_specs=[
            pl.BlockSpec((1, 1, RB), lambda i: (i, 0, 0)),
            pl.BlockSpec((RB, D), lambda i: (i, 0)),
        ],
        out_specs=[
            pl.BlockSpec((B, D), lambda i: (0, 0)),
            pl.BlockSpec((1, B), lambda i: (0, 0)),
        ],
        out_shape=[
            jax.ShapeDtypeStruct((B, D), jnp.float32),
            jax.ShapeDtypeStruct((1, B), jnp.float32),
        ],
    )(batch3, x)

    row = lambda v: v.reshape(1, D)
    z, vn_out = pl.pallas_call(
        functools.partial(_mlp_kernel, N),
        out_shape=[
            jax.ShapeDtypeStruct((B, D), jnp.float32),
            jax.ShapeDtypeStruct((B, D), jnp.float32),
        ],
    )(sums, counts, vn_embedding,
      W_node2vn, row(b_node2vn), row(g_node2vn), row(be_node2vn),
      W_vn2node, row(b_vn2node), row(g_vn2node), row(be_vn2node))

    x_out = pl.pallas_call(
        _bcast_kernel,
        grid=(nblk,),
        in_specs=[
            pl.BlockSpec((1, 1, RB), lambda i: (i, 0, 0)),
            pl.BlockSpec((RB, D), lambda i: (i, 0)),
            pl.BlockSpec((B, D), lambda i: (0, 0)),
        ],
        out_specs=pl.BlockSpec((RB, D), lambda i: (i, 0)),
        out_shape=jax.ShapeDtypeStruct((N, D), jnp.float32),
    )(batch3, x, z)

    return (x_out, vn_out)
